# baseline (speedup 1.0000x reference)
import functools

import jax
import jax.numpy as jnp
from jax import lax
from jax.experimental import pallas as pl
from jax.experimental.pallas import tpu as pltpu
from jax.experimental.pallas import tpu_sc as plsc

_NC, _NS = 2, 16
_NW = _NC * _NS


def _routing_metadata(top_k_index, top_k_weights, num_experts, bm, nblk, pad):
    tk, k = top_k_index.shape
    n = tk * k
    flat_e = top_k_index.reshape(-1).astype(jnp.int32)
    onehot = (flat_e[:, None] == jnp.arange(num_experts, dtype=jnp.int32)[None, :]).astype(jnp.int32)
    csum = jnp.cumsum(onehot, axis=0)
    counts = csum[-1]
    rank = jnp.take_along_axis(csum, flat_e[:, None], axis=1)[:, 0] - 1
    nblk_e = (counts + bm - 1) // bm
    start_blk = jnp.cumsum(nblk_e) - nblk_e
    dest = start_blk[flat_e] * bm + rank
    num_active = jnp.sum(nblk_e).astype(jnp.int32)
    bid = jnp.arange(nblk, dtype=jnp.int32)
    be = (jnp.searchsorted(start_blk, bid, side="right") - 1).astype(jnp.int32)
    be = jnp.where(bid < num_active, be, jnp.take(be, num_active - 1))
    tok = (jnp.arange(n, dtype=jnp.int32) // k)
    tok_pad = jnp.zeros((pad,), jnp.int32).at[dest].set(tok)
    w_pad = jnp.zeros((pad,), jnp.float32).at[dest].set(
        top_k_weights.reshape(-1).astype(jnp.float32))
    meta = jnp.concatenate([be, num_active[None]])
    gidx = dest.reshape(tk, k)
    return tok_pad, w_pad[:, None], meta, gidx


def _sc_gather(hidden_states, tok_pad, pad, h):
    rpw = pad // _NW
    ch = 8
    nch = rpw // ch
    tok3 = tok_pad.reshape(_NW, nch, ch)
    mesh = plsc.VectorSubcoreMesh(core_axis_name="c", subcore_axis_name="s")

    @functools.partial(
        pl.kernel, mesh=mesh,
        out_type=jax.ShapeDtypeStruct((pad, h), jnp.float32),
        scratch_types=[
            pltpu.VMEM((nch, ch), jnp.int32),
            pltpu.VMEM((ch, h), jnp.float32),
            pltpu.VMEM((ch, h), jnp.float32),
            pltpu.SemaphoreType.DMA,
            pltpu.SemaphoreType.DMA,
        ],
    )
    def k(hs_hbm, tok_hbm, xs_hbm, idx_v, buf0, buf1, sem0, sem1):
        wid = lax.axis_index("s") * _NC + lax.axis_index("c")
        base = wid * rpw
        pltpu.sync_copy(tok_hbm.at[wid], idx_v)
        bufs = (buf0, buf1)
        sems = (sem0, sem1)
        pltpu.async_copy(hs_hbm.at[idx_v.at[0]], bufs[0], sems[0])
        for j in range(nch):
            if j + 1 < nch:
                pltpu.async_copy(hs_hbm.at[idx_v.at[j + 1]], bufs[(j + 1) % 2],
                                 sems[(j + 1) % 2])
            pltpu.make_async_copy(hs_hbm.at[idx_v.at[j]], bufs[j % 2],
                                  sems[j % 2]).wait()
            pltpu.sync_copy(bufs[j % 2], xs_hbm.at[pl.ds(base + j * ch, ch)])

    return k(hidden_states, tok3)


def _tc_ffn(x_sorted, w_pad, meta, gate_up_proj, down_proj, bm, nblk, pad):
    e, h, i2 = gate_up_proj.shape
    i = i2 // 2

    def body(meta_ref, w_ref, x_ref, gu_ref, dp_ref, out_ref):
        b = pl.program_id(0)

        @pl.when(b < meta_ref[nblk])
        def _():
            x = x_ref[...].astype(jnp.bfloat16)
            gu = jnp.dot(x, gu_ref[0].astype(jnp.bfloat16),
                         preferred_element_type=jnp.float32)
            gate = gu[:, :i]
            up = gu[:, i:]
            act = gate * jax.nn.sigmoid(gate) * up * w_ref[...]
            out_ref[...] = jnp.dot(act.astype(jnp.bfloat16),
                                   dp_ref[0].astype(jnp.bfloat16),
                                   preferred_element_type=jnp.float32)

    grid_spec = pltpu.PrefetchScalarGridSpec(
        num_scalar_prefetch=1,
        grid=(nblk,),
        in_specs=[
            pl.BlockSpec((bm, 1), lambda b, m: (b, 0)),
            pl.BlockSpec((bm, h), lambda b, m: (b, 0)),
            pl.BlockSpec((1, h, i2), lambda b, m: (m[b], 0, 0)),
            pl.BlockSpec((1, i, h), lambda b, m: (m[b], 0, 0)),
        ],
        out_specs=pl.BlockSpec((bm, h), lambda b, m: (b, 0)),
    )
    return pl.pallas_call(
        body,
        grid_spec=grid_spec,
        out_shape=jax.ShapeDtypeStruct((pad, h), jnp.float32),
    )(meta, w_pad, x_sorted, gate_up_proj, down_proj)


def _sc_combine(part, gidx, t, h):
    tpw = t // _NW
    ch = 16
    nch = tpw // ch
    g0 = gidx[:, 0].reshape(_NW, nch, ch)
    g1 = gidx[:, 1].reshape(_NW, nch, ch)
    mesh = plsc.VectorSubcoreMesh(core_axis_name="c", subcore_axis_name="s")
    nvec = ch * (h // 16)
    cshift = 0
    hh = h // 16
    while (1 << cshift) < hh:
        cshift += 1

    @functools.partial(
        pl.kernel, mesh=mesh,
        out_type=jax.ShapeDtypeStruct((t, h), jnp.float32),
        scratch_types=[
            pltpu.VMEM((nch, ch), jnp.int32),
            pltpu.VMEM((nch, ch), jnp.int32),
            pltpu.VMEM((ch, h), jnp.float32),
            pltpu.VMEM((ch, h), jnp.float32),
            pltpu.SemaphoreType.DMA,
            pltpu.SemaphoreType.DMA,
        ],
    )
    def k(part_hbm, g0_hbm, g1_hbm, out_hbm, i0, i1, ba, bb, sa, sb):
        wid = lax.axis_index("s") * _NC + lax.axis_index("c")
        base = wid * tpw
        pltpu.sync_copy(g0_hbm.at[wid], i0)
        pltpu.sync_copy(g1_hbm.at[wid], i1)
        for j in range(nch):
            ca = pltpu.async_copy(part_hbm.at[i0.at[j]], ba, sa)
            cb = pltpu.async_copy(part_hbm.at[i1.at[j]], bb, sb)
            ca.wait()
            cb.wait()

            def add_body(tt, carry):
                r = lax.shift_right_logical(tt, cshift)
                c = pl.multiple_of(lax.shift_left(lax.bitwise_and(tt, hh - 1), 4), 16)
                ba[r, pl.ds(c, 16)] = ba[r, pl.ds(c, 16)] + bb[r, pl.ds(c, 16)]
                return carry

            lax.fori_loop(0, nvec, add_body, 0, unroll=4)
            pltpu.sync_copy(ba, out_hbm.at[pl.ds(base + j * ch, ch)])

    return k(part, g0, g1)


def kernel(hidden_states, top_k_index, top_k_weights, gate_up_proj, down_proj):
    t, h = hidden_states.shape
    e = gate_up_proj.shape[0]
    k = top_k_index.shape[1]
    bm = 256
    n = t * k
    nblk = n // bm + e - 1
    pad = nblk * bm

    tok_pad, w_pad, meta, gidx = _routing_metadata(
        top_k_index, top_k_weights, e, bm, nblk, pad)
    x_sorted = _sc_gather(hidden_states, tok_pad, pad, h)
    part = _tc_ffn(x_sorted, w_pad, meta, gate_up_proj, down_proj, bm, nblk, pad)
    return _sc_combine(part, gidx, t, h)
